# baseline (speedup 1.0000x reference)
import functools

import jax
import jax.numpy as jnp
from jax import lax
from jax.experimental import pallas as pl
from jax.experimental.pallas import tpu as pltpu
from jax.experimental.pallas import tpu_sc as plsc

N = 10000
E = 320000
D = 128

NC = 2
NS = 16
NW = NC * NS

EPW = E // NW
K = 80
C = EPW // K
KM = 100
CM = EPW // KM
RING = 2

_mesh = plsc.VectorSubcoreMesh(core_axis_name="c", subcore_axis_name="s")
_sc_params = pltpu.CompilerParams(use_tc_tiling_on_sc=False)


@functools.partial(
    pl.kernel,
    out_type=jax.ShapeDtypeStruct((NC * N,), jnp.float32),
    mesh=_mesh,
    compiler_params=_sc_params,
    scratch_types=[
        pltpu.VMEM((CM, KM), jnp.int32),
        pltpu.VMEM((112,), jnp.float32),
        pltpu.VMEM_SHARED((N,), jnp.float32),
    ],
)
def _deg_kernel(dst_hbm, z1_hbm, deg_out, dst_v, ones_v, acc):
    c = lax.axis_index("c")
    s = lax.axis_index("s")
    w = c * NS + s
    pltpu.sync_copy(dst_hbm.at[w], dst_v)
    for i in range(112 // 16):
        ones_v[pl.ds(i * 16, 16)] = jnp.full((16,), 1.0, jnp.float32)
    @pl.when(s < 10)
    def _():
        off = pl.multiple_of(s * 1000, 1000)
        pltpu.sync_copy(z1_hbm, acc.at[pl.ds(off, 1000)])
    plsc.subcore_barrier()

    def body(j, carry):
        pltpu.sync_copy(ones_v.at[pl.ds(0, KM)], acc.at[dst_v.at[j]],
                        add=True)
        return carry

    lax.fori_loop(0, CM, body, 0)
    plsc.subcore_barrier()

    @pl.when(s < 10)
    def _():
        off = pl.multiple_of(s * 1000, 1000)
        offo = pl.multiple_of(c * N + s * 1000, 1000)
        pltpu.sync_copy(acc.at[pl.ds(off, 1000)], deg_out.at[pl.ds(offo, 1000)])


@functools.partial(
    pl.kernel,
    out_type=jax.ShapeDtypeStruct((NC, N, D), jnp.float32),
    mesh=_mesh,
    compiler_params=_sc_params,
    scratch_types=[
        pltpu.VMEM((CM, KM), jnp.int32),
        pltpu.VMEM((CM, KM), jnp.int32),
        pltpu.VMEM((RING, KM, D), jnp.float32),
        pltpu.VMEM_SHARED((N, D), jnp.float32),
        [pltpu.SemaphoreType.DMA] * RING,
        [pltpu.SemaphoreType.DMA] * RING,
    ],
)
def _msg_kernel(g_hbm, src_hbm, dst_hbm, z2_hbm, q_out,
                src_v, dst_v, rows_v, acc, gsems, ssems):
    c = lax.axis_index("c")
    s = lax.axis_index("s")
    w = c * NS + s
    pltpu.sync_copy(src_hbm.at[w], src_v)
    pltpu.sync_copy(dst_hbm.at[w], dst_v)
    @pl.when(s < 10)
    def _():
        for k2 in range(5):
            off = pl.multiple_of(s * 1000 + k2 * 200, 200)
            pltpu.sync_copy(z2_hbm, acc.at[pl.ds(off, 200)])
    plsc.subcore_barrier()

    def gissue(q, b):
        pltpu.async_copy(g_hbm.at[src_v.at[q]], rows_v.at[b], gsems[b])

    def gwait(q, b):
        pltpu.make_async_copy(g_hbm.at[src_v.at[q]], rows_v.at[b],
                              gsems[b]).wait()

    def sissue(q, b):
        pltpu.async_copy(rows_v.at[b], acc.at[dst_v.at[q]], ssems[b],
                         add=True)

    def swait(q, b):
        pltpu.make_async_copy(rows_v.at[b], acc.at[dst_v.at[q]],
                              ssems[b]).wait()

    for b in range(RING):
        gissue(b, b)

    def ring_body(j2, carry):
        j = j2 * RING
        for b in range(RING):
            gwait(j + b, b)
            sissue(j + b, b)
        for b in range(RING):
            @pl.when(j + RING + b < CM)
            def _():
                swait(j + b, b)
                gissue(j + RING + b, b)
        return carry

    lax.fori_loop(0, CM // RING, ring_body, 0)
    for b in range(RING):
        swait(CM - RING + b, b)
    plsc.subcore_barrier()

    @pl.when(s < 10)
    def _():
        for k2 in range(5):
            off = pl.multiple_of(s * 1000 + k2 * 200, 200)
            pltpu.sync_copy(acc.at[pl.ds(off, 200)], q_out.at[c, pl.ds(off, 200)])


ROWS_BLK = 1000


def _lin_body(x_ref, w_ref, degp_ref, g_ref, dis_ref):
    deg = degp_ref[:, 0] + degp_ref[:, 1]
    dis = jnp.where(deg > 0, lax.rsqrt(jnp.maximum(deg, 1e-12)), 0.0)
    h = lax.dot_general(x_ref[...], w_ref[...],
                        (((1,), (1,)), ((), ())),
                        preferred_element_type=jnp.float32)
    g_ref[...] = h * dis[:, None]
    dis_ref[...] = dis[:, None]


def _linear_scale(x, W, deg_partials):
    grid = N // ROWS_BLK
    return pl.pallas_call(
        _lin_body,
        grid=(grid,),
        in_specs=[
            pl.BlockSpec((ROWS_BLK, D), lambda i: (i, 0)),
            pl.BlockSpec((D, D), lambda i: (0, 0)),
            pl.BlockSpec((ROWS_BLK, NC), lambda i: (i, 0)),
        ],
        out_specs=[
            pl.BlockSpec((ROWS_BLK, D), lambda i: (i, 0)),
            pl.BlockSpec((ROWS_BLK, 1), lambda i: (i, 0)),
        ],
        out_shape=[
            jax.ShapeDtypeStruct((N, D), jnp.float32),
            jax.ShapeDtypeStruct((N, 1), jnp.float32),
        ],
    )(x, W, deg_partials)


def _fin_body(q_ref, dis_ref, b_ref, o_ref):
    q = q_ref[0] + q_ref[1]
    o_ref[...] = jnp.maximum(q * dis_ref[...] + b_ref[...], 0.0)


def _finalize(q_partials, dis, b2):
    grid = N // ROWS_BLK
    return pl.pallas_call(
        _fin_body,
        grid=(grid,),
        in_specs=[
            pl.BlockSpec((NC, ROWS_BLK, D), lambda i: (0, i, 0)),
            pl.BlockSpec((ROWS_BLK, 1), lambda i: (i, 0)),
            pl.BlockSpec((1, D), lambda i: (0, 0)),
        ],
        out_specs=pl.BlockSpec((ROWS_BLK, D), lambda i: (i, 0)),
        out_shape=jax.ShapeDtypeStruct((N, D), jnp.float32),
    )(q_partials, dis, b2)


def kernel(x, edge_index, W, b):
    ei = edge_index.astype(jnp.int32)
    src3 = ei[0].reshape(NW, CM, KM)
    dst3 = ei[1].reshape(NW, CM, KM)
    z1 = jnp.zeros((1000,), jnp.float32)
    z2 = jnp.zeros((200, D), jnp.float32)
    b2 = b.reshape(1, D).astype(jnp.float32)

    deg_partials = _deg_kernel(dst3, z1)
    g, dis = _linear_scale(x, W, deg_partials.reshape(NC, N).T)
    q_partials = _msg_kernel(g, src3, dst3, z2)
    return _finalize(q_partials, dis, b2)


# speedup vs baseline: 1.1888x; 1.1888x over previous
import functools

import jax
import jax.numpy as jnp
from jax import lax
from jax.experimental import pallas as pl
from jax.experimental.pallas import tpu as pltpu
from jax.experimental.pallas import tpu_sc as plsc

N = 10000
E = 320000
D = 128

NC = 2
NS = 16
NW = NC * NS

EPW = E // NW
K = 80
C = EPW // K
KM = 100
CM = EPW // KM
RING = 2

_mesh = plsc.VectorSubcoreMesh(core_axis_name="c", subcore_axis_name="s")
_sc_params = pltpu.CompilerParams(use_tc_tiling_on_sc=False)


@functools.partial(
    pl.kernel,
    out_type=jax.ShapeDtypeStruct((NC * N,), jnp.float32),
    mesh=_mesh,
    compiler_params=_sc_params,
    scratch_types=[
        pltpu.VMEM((CM, KM), jnp.int32),
        pltpu.VMEM((112,), jnp.float32),
        pltpu.VMEM_SHARED((N,), jnp.float32),
    ],
)
def _deg_kernel(dst_hbm, z1_hbm, deg_out, dst_v, ones_v, acc):
    c = lax.axis_index("c")
    s = lax.axis_index("s")
    w = c * NS + s
    pltpu.sync_copy(dst_hbm.at[w], dst_v)
    for i in range(112 // 16):
        ones_v[pl.ds(i * 16, 16)] = jnp.full((16,), 1.0, jnp.float32)
    @pl.when(s < 10)
    def _():
        off = pl.multiple_of(s * 1000, 1000)
        pltpu.sync_copy(z1_hbm, acc.at[pl.ds(off, 1000)])
    plsc.subcore_barrier()

    def body(j, carry):
        pltpu.sync_copy(ones_v.at[pl.ds(0, KM)], acc.at[dst_v.at[j]],
                        add=True)
        return carry

    lax.fori_loop(0, CM, body, 0)
    plsc.subcore_barrier()

    @pl.when(s < 10)
    def _():
        off = pl.multiple_of(s * 1000, 1000)
        offo = pl.multiple_of(c * N + s * 1000, 1000)
        pltpu.sync_copy(acc.at[pl.ds(off, 1000)], deg_out.at[pl.ds(offo, 1000)])


@functools.partial(
    pl.kernel,
    out_type=jax.ShapeDtypeStruct((NC, N, D), jnp.float32),
    mesh=_mesh,
    compiler_params=_sc_params,
    scratch_types=[
        pltpu.VMEM((CM, KM), jnp.int32),
        pltpu.VMEM((CM, KM), jnp.int32),
        pltpu.VMEM((RING, KM, D), jnp.float32),
        pltpu.VMEM_SHARED((N, D), jnp.float32),
        [pltpu.SemaphoreType.DMA] * RING,
        [pltpu.SemaphoreType.DMA] * RING,
    ],
)
def _msg_kernel(g_hbm, src_hbm, dst_hbm, z2_hbm, q_out,
                src_v, dst_v, rows_v, acc, gsems, ssems):
    c = lax.axis_index("c")
    s = lax.axis_index("s")
    w = c * NS + s
    pltpu.sync_copy(src_hbm.at[w], src_v)
    pltpu.sync_copy(dst_hbm.at[w], dst_v)
    @pl.when(s < 10)
    def _():
        for k2 in range(5):
            off = pl.multiple_of(s * 1000 + k2 * 200, 200)
            pltpu.sync_copy(z2_hbm, acc.at[pl.ds(off, 200)])
    plsc.subcore_barrier()

    def gissue(q, b):
        pltpu.async_copy(g_hbm.at[src_v.at[q]], rows_v.at[b], gsems[b])

    def gwait(q, b):
        pltpu.make_async_copy(g_hbm.at[src_v.at[q]], rows_v.at[b],
                              gsems[b]).wait()

    def sissue(q, b):
        pltpu.async_copy(rows_v.at[b], acc.at[dst_v.at[q]], ssems[b],
                         add=True)

    def swait(q, b):
        pltpu.make_async_copy(rows_v.at[b], acc.at[dst_v.at[q]],
                              ssems[b]).wait()

    for b in range(RING):
        gissue(b, b)

    def ring_body(j2, carry):
        j = j2 * RING
        for b in range(RING):
            gwait(j + b, b)
            pltpu.sync_copy(rows_v.at[b], acc.at[dst_v.at[j + b]], add=True)

            @pl.when(j + RING + b < CM)
            def _():
                gissue(j + RING + b, b)
        return carry

    lax.fori_loop(0, CM // RING, ring_body, 0)
    plsc.subcore_barrier()

    @pl.when(s < 10)
    def _():
        for k2 in range(5):
            off = pl.multiple_of(s * 1000 + k2 * 200, 200)
            pltpu.sync_copy(acc.at[pl.ds(off, 200)], q_out.at[c, pl.ds(off, 200)])


ROWS_BLK = 1000


def _lin_body(x_ref, w_ref, degp_ref, g_ref, dis_ref):
    deg = degp_ref[:, 0] + degp_ref[:, 1]
    dis = jnp.where(deg > 0, lax.rsqrt(jnp.maximum(deg, 1e-12)), 0.0)
    h = lax.dot_general(x_ref[...], w_ref[...],
                        (((1,), (1,)), ((), ())),
                        preferred_element_type=jnp.float32)
    g_ref[...] = h * dis[:, None]
    dis_ref[...] = dis[:, None]


def _linear_scale(x, W, deg_partials):
    grid = N // ROWS_BLK
    return pl.pallas_call(
        _lin_body,
        grid=(grid,),
        in_specs=[
            pl.BlockSpec((ROWS_BLK, D), lambda i: (i, 0)),
            pl.BlockSpec((D, D), lambda i: (0, 0)),
            pl.BlockSpec((ROWS_BLK, NC), lambda i: (i, 0)),
        ],
        out_specs=[
            pl.BlockSpec((ROWS_BLK, D), lambda i: (i, 0)),
            pl.BlockSpec((ROWS_BLK, 1), lambda i: (i, 0)),
        ],
        out_shape=[
            jax.ShapeDtypeStruct((N, D), jnp.float32),
            jax.ShapeDtypeStruct((N, 1), jnp.float32),
        ],
    )(x, W, deg_partials)


def _fin_body(q_ref, dis_ref, b_ref, o_ref):
    q = q_ref[0] + q_ref[1]
    o_ref[...] = jnp.maximum(q * dis_ref[...] + b_ref[...], 0.0)


def _finalize(q_partials, dis, b2):
    grid = N // ROWS_BLK
    return pl.pallas_call(
        _fin_body,
        grid=(grid,),
        in_specs=[
            pl.BlockSpec((NC, ROWS_BLK, D), lambda i: (0, i, 0)),
            pl.BlockSpec((ROWS_BLK, 1), lambda i: (i, 0)),
            pl.BlockSpec((1, D), lambda i: (0, 0)),
        ],
        out_specs=pl.BlockSpec((ROWS_BLK, D), lambda i: (i, 0)),
        out_shape=jax.ShapeDtypeStruct((N, D), jnp.float32),
    )(q_partials, dis, b2)


def kernel(x, edge_index, W, b):
    ei = edge_index.astype(jnp.int32)
    src3 = ei[0].reshape(NW, CM, KM)
    dst3 = ei[1].reshape(NW, CM, KM)
    z1 = jnp.zeros((1000,), jnp.float32)
    z2 = jnp.zeros((200, D), jnp.float32)
    b2 = b.reshape(1, D).astype(jnp.float32)

    deg_partials = _deg_kernel(dst3, z1)
    g, dis = _linear_scale(x, W, deg_partials.reshape(NC, N).T)
    q_partials = _msg_kernel(g, src3, dst3, z2)
    return _finalize(q_partials, dis, b2)


# speedup vs baseline: 1.2304x; 1.0349x over previous
import functools

import jax
import jax.numpy as jnp
from jax import lax
from jax.experimental import pallas as pl
from jax.experimental.pallas import tpu as pltpu
from jax.experimental.pallas import tpu_sc as plsc

N = 10000
E = 320000
D = 128

NC = 2
NS = 16
NW = NC * NS

EPW = E // NW
K = 80
C = EPW // K
KM = 100
CM = EPW // KM
RING = 2

_mesh = plsc.VectorSubcoreMesh(core_axis_name="c", subcore_axis_name="s")
_sc_params = pltpu.CompilerParams(use_tc_tiling_on_sc=False)


@functools.partial(
    pl.kernel,
    out_type=jax.ShapeDtypeStruct((NC * N,), jnp.float32),
    mesh=_mesh,
    compiler_params=_sc_params,
    scratch_types=[
        pltpu.VMEM((CM, KM), jnp.int32),
        pltpu.VMEM((112,), jnp.float32),
        pltpu.VMEM_SHARED((N,), jnp.float32),
    ],
)
def _deg_kernel(dst_hbm, z1_hbm, deg_out, dst_v, ones_v, acc):
    c = lax.axis_index("c")
    s = lax.axis_index("s")
    w = c * NS + s
    pltpu.sync_copy(dst_hbm.at[w], dst_v)
    for i in range(112 // 16):
        ones_v[pl.ds(i * 16, 16)] = jnp.full((16,), 1.0, jnp.float32)
    @pl.when(s < 10)
    def _():
        off = pl.multiple_of(s * 1000, 1000)
        pltpu.sync_copy(z1_hbm, acc.at[pl.ds(off, 1000)])
    plsc.subcore_barrier()

    def body(j, carry):
        pltpu.sync_copy(ones_v.at[pl.ds(0, KM)], acc.at[dst_v.at[j]],
                        add=True)
        return carry

    lax.fori_loop(0, CM, body, 0)
    plsc.subcore_barrier()

    @pl.when(s < 10)
    def _():
        off = pl.multiple_of(s * 1000, 1000)
        offo = pl.multiple_of(c * N + s * 1000, 1000)
        pltpu.sync_copy(acc.at[pl.ds(off, 1000)], deg_out.at[pl.ds(offo, 1000)])


@functools.partial(
    pl.kernel,
    out_type=jax.ShapeDtypeStruct((NC, N, D), jnp.float32),
    mesh=_mesh,
    compiler_params=_sc_params,
    scratch_types=[
        pltpu.VMEM((CM, KM), jnp.int32),
        pltpu.VMEM((CM, KM), jnp.int32),
        pltpu.VMEM((RING, KM, D), jnp.float32),
        pltpu.VMEM_SHARED((N, D), jnp.float32),
        [pltpu.SemaphoreType.DMA] * RING,
    ],
)
def _msg_kernel(g_hbm, src_hbm, dst_hbm, z2_hbm, q_out,
                src_v, dst_v, rows_v, acc, gsems):
    c = lax.axis_index("c")
    s = lax.axis_index("s")
    w = c * NS + s
    pltpu.sync_copy(src_hbm.at[w], src_v)
    pltpu.sync_copy(dst_hbm.at[w], dst_v)

    def gissue(q, b):
        pltpu.async_copy(g_hbm.at[src_v.at[q]], rows_v.at[b], gsems[b])

    def gwait(q, b):
        pltpu.make_async_copy(g_hbm.at[src_v.at[q]], rows_v.at[b],
                              gsems[b]).wait()

    for b in range(RING):
        gissue(b, b)
    @pl.when(s < 10)
    def _():
        off = pl.multiple_of(s * 1000, 1000)
        pltpu.sync_copy(z2_hbm, acc.at[pl.ds(off, 1000)])
    plsc.subcore_barrier()

    def ring_body(j2, carry):
        j = j2 * RING
        for b in range(RING):
            gwait(j + b, b)
            pltpu.sync_copy(rows_v.at[b], acc.at[dst_v.at[j + b]], add=True)

            @pl.when(j + RING + b < CM)
            def _():
                gissue(j + RING + b, b)
        return carry

    lax.fori_loop(0, CM // RING, ring_body, 0)
    plsc.subcore_barrier()

    @pl.when(s < 10)
    def _():
        off = pl.multiple_of(s * 1000, 1000)
        pltpu.sync_copy(acc.at[pl.ds(off, 1000)], q_out.at[c, pl.ds(off, 1000)])


ROWS_BLK = 1000


def _lin_body(x_ref, w_ref, degp_ref, g_ref, dis_ref):
    deg = degp_ref[:, 0] + degp_ref[:, 1]
    dis = jnp.where(deg > 0, lax.rsqrt(jnp.maximum(deg, 1e-12)), 0.0)
    h = lax.dot_general(x_ref[...], w_ref[...],
                        (((1,), (1,)), ((), ())),
                        preferred_element_type=jnp.float32)
    g_ref[...] = h * dis[:, None]
    dis_ref[...] = dis[:, None]


def _linear_scale(x, W, deg_partials):
    grid = N // ROWS_BLK
    return pl.pallas_call(
        _lin_body,
        grid=(grid,),
        in_specs=[
            pl.BlockSpec((ROWS_BLK, D), lambda i: (i, 0)),
            pl.BlockSpec((D, D), lambda i: (0, 0)),
            pl.BlockSpec((ROWS_BLK, NC), lambda i: (i, 0)),
        ],
        out_specs=[
            pl.BlockSpec((ROWS_BLK, D), lambda i: (i, 0)),
            pl.BlockSpec((ROWS_BLK, 1), lambda i: (i, 0)),
        ],
        out_shape=[
            jax.ShapeDtypeStruct((N, D), jnp.float32),
            jax.ShapeDtypeStruct((N, 1), jnp.float32),
        ],
    )(x, W, deg_partials)


def _fin_body(q_ref, dis_ref, b_ref, o_ref):
    q = q_ref[0] + q_ref[1]
    o_ref[...] = jnp.maximum(q * dis_ref[...] + b_ref[...], 0.0)


def _finalize(q_partials, dis, b2):
    grid = N // ROWS_BLK
    return pl.pallas_call(
        _fin_body,
        grid=(grid,),
        in_specs=[
            pl.BlockSpec((NC, ROWS_BLK, D), lambda i: (0, i, 0)),
            pl.BlockSpec((ROWS_BLK, 1), lambda i: (i, 0)),
            pl.BlockSpec((1, D), lambda i: (0, 0)),
        ],
        out_specs=pl.BlockSpec((ROWS_BLK, D), lambda i: (i, 0)),
        out_shape=jax.ShapeDtypeStruct((N, D), jnp.float32),
    )(q_partials, dis, b2)


def kernel(x, edge_index, W, b):
    ei = edge_index.astype(jnp.int32)
    src3 = ei[0].reshape(NW, CM, KM)
    dst3 = ei[1].reshape(NW, CM, KM)
    z1 = jnp.zeros((1000,), jnp.float32)
    z2 = jnp.zeros((1000, D), jnp.float32)
    b2 = b.reshape(1, D).astype(jnp.float32)

    deg_partials = _deg_kernel(dst3, z1)
    g, dis = _linear_scale(x, W, deg_partials.reshape(NC, N).T)
    q_partials = _msg_kernel(g, src3, dst3, z2)
    return _finalize(q_partials, dis, b2)
